# Initial kernel scaffold; baseline (speedup 1.0000x reference)
import jax
import jax.numpy as jnp
from jax import lax
from jax.experimental import pallas as pl
from jax.experimental.pallas import tpu as pltpu
from jax.experimental.pallas import tpu_sc as plsc

E = 8
TOPK = 2
D = 768
DFF = 2688
T = 2048

R = 256
CAP = T * TOPK + E * R
G_MAX = CAP // R
F = 3
DFB = DFF // F
EPAD = 128

NW = 32
TPW = T // NW
NB_SLOT = G_MAX


def _lane16():
    return lax.iota(jnp.int32, 16)


def _vlane(v, e):
    return jnp.sum(jnp.where(_lane16() == e, v, jnp.zeros_like(v)))


def _router_body(x_ref, wg_ref, o_ref):
    lg = jnp.dot(x_ref[...], wg_ref[...], preferred_element_type=jnp.float32)
    lane = lax.broadcasted_iota(jnp.int32, lg.shape, 1)
    valid = lane < E
    lgm = jnp.where(valid, lg, jnp.float32(-1e30))
    m = jnp.max(lgm, axis=1, keepdims=True)
    ex = jnp.where(valid, jnp.exp(lgm - m), 0.0)
    p = ex / jnp.sum(ex, axis=1, keepdims=True)
    m1 = jnp.max(p, axis=1, keepdims=True)
    i1 = jnp.min(jnp.where(p >= m1, lane, EPAD), axis=1, keepdims=True)
    p2 = jnp.where(lane == i1, jnp.float32(-1.0), p)
    m2 = jnp.max(p2, axis=1, keepdims=True)
    i2 = jnp.min(jnp.where(p2 >= m2, lane, EPAD), axis=1, keepdims=True)
    s = m1 + m2
    o_ref[...] = jnp.where(
        lane == 0, i1.astype(jnp.float32),
        jnp.where(lane == 1, i2.astype(jnp.float32),
                  jnp.where(lane == 2, m1 / s,
                            jnp.where(lane == 3, m2 / s, 0.0))))


def _router(x, wg_pad):
    return pl.pallas_call(
        _router_body,
        grid=(T // R,),
        in_specs=[
            pl.BlockSpec((R, D), lambda i: (i, 0)),
            pl.BlockSpec((D, EPAD), lambda i: (0, 0)),
        ],
        out_specs=pl.BlockSpec((R, EPAD), lambda i: (i, 0)),
        out_shape=jax.ShapeDtypeStruct((T, EPAD), jnp.float32),
    )(x, wg_pad)


def _dispatch_body(x_hbm, i0_hbm, i1_hbm,
                   xs_hbm, pos0_hbm, pos1_hbm, blk_hbm,
                   ivm0, ivm1, xrows, p0v, p1v, blkv):
    cid = lax.axis_index("c")
    sid = lax.axis_index("s")
    wid = sid * 2 + cid
    mybase = wid * TPW
    zeros16 = jnp.zeros((16,), jnp.int32)
    lanes = _lane16()

    pltpu.sync_copy(i0_hbm, ivm0)
    pltpu.sync_copy(i1_hbm, ivm1)
    pltpu.sync_copy(x_hbm.at[pl.ds(mybase, TPW)], xrows)

    def chunk_step(w, carry):
        totals, prefix = carry
        wv = jnp.full((16,), w, jnp.int32)
        is_prev = wv < jnp.full((16,), wid, jnp.int32)
        base = w * TPW
        for r in range(TPW // 16):
            for src in (ivm0, ivm1):
                v = src[pl.ds(base + r * 16, 16)]
                for e in range(E):
                    cnt = plsc.all_reduce_population_count(v == e)
                    delta = jnp.where(lanes == e, cnt, zeros16)
                    totals = totals + delta
                    prefix = prefix + jnp.where(is_prev, delta, zeros16)
        return totals, prefix

    totals, prefix = lax.fori_loop(0, NW, chunk_step, (zeros16, zeros16))

    padded = ((totals + (R - 1)) // R) * R
    incl = plsc.cumsum(padded)
    seg_start = incl - padded
    nb = _vlane(incl, E - 1) // R

    running = seg_start + prefix
    for r in range(TPW // 16):
        for src, dst in ((ivm0, p0v), (ivm1, p1v)):
            v = src[pl.ds(mybase + r * 16, 16)]
            posv = zeros16
            for e in range(E):
                m = v == e
                ones = jnp.where(m, jnp.ones((16,), jnp.int32), zeros16)
                rank = plsc.cumsum(ones) - 1
                b_e = _vlane(running, e)
                posv = jnp.where(m, b_e + rank, posv)
                running = running + jnp.where(lanes == e, jnp.sum(ones), 0)
            dst[0, pl.ds(r * 16, 16)] = posv

    pltpu.sync_copy(p0v.at[0], pos0_hbm.at[pl.ds(mybase, TPW)])
    pltpu.sync_copy(p1v.at[0], pos1_hbm.at[pl.ds(mybase, TPW)])
    pltpu.sync_copy(xrows, xs_hbm.at[p0v.at[0]])
    pltpu.sync_copy(xrows, xs_hbm.at[p1v.at[0]])

    @pl.when(wid == 0)
    def _():
        for half in range(2):
            bidx = lanes + half * 16
            beff = jnp.minimum(bidx, nb - 1)
            posb = beff * R
            ev = zeros16
            for e in range(E):
                s_e = _vlane(seg_start, e)
                p_e = _vlane(padded, e)
                m = (posb >= s_e) & (posb < s_e + p_e)
                ev = jnp.where(m, jnp.full((16,), e, jnp.int32), ev)
            if half == 1:
                ev = jnp.where(lanes == NB_SLOT - 16,
                               jnp.full((16,), nb, jnp.int32), ev)
            blkv[0, pl.ds(half * 16, 16)] = ev
        pltpu.sync_copy(blkv.at[0], blk_hbm)


def _dispatch(x, i0, i1):
    mesh = plsc.VectorSubcoreMesh(core_axis_name="c", subcore_axis_name="s")
    return pl.kernel(
        _dispatch_body,
        out_type=(
            jax.ShapeDtypeStruct((CAP, D), jnp.float32),
            jax.ShapeDtypeStruct((T,), jnp.int32),
            jax.ShapeDtypeStruct((T,), jnp.int32),
            jax.ShapeDtypeStruct((32,), jnp.int32),
        ),
        mesh=mesh,
        scratch_types=[
            pltpu.VMEM((T,), jnp.int32),
            pltpu.VMEM((T,), jnp.int32),
            pltpu.VMEM((TPW, D), jnp.float32),
            pltpu.VMEM((1, TPW), jnp.int32),
            pltpu.VMEM((1, TPW), jnp.int32),
            pltpu.VMEM((1, 32), jnp.int32),
        ],
    )(x, i0, i1)


def _ffn_body(s_ref, xs_ref, w1_ref, w3_ref, w2_ref, ys_ref, acc_ref):
    f = pl.program_id(0)
    b = pl.program_id(1)

    @pl.when(b < s_ref[NB_SLOT])
    def _():
        xb = xs_ref[...].astype(jnp.bfloat16)
        w1e = w1_ref[0].astype(jnp.bfloat16)
        w3e = w3_ref[0].astype(jnp.bfloat16)
        w2e = w2_ref[0].astype(jnp.bfloat16)
        nt = (((1,), (1,)), ((), ()))
        h1 = lax.dot_general(xb, w1e, nt, preferred_element_type=jnp.float32)
        h3 = lax.dot_general(xb, w3e, nt, preferred_element_type=jnp.float32)
        g = (h1 * jax.nn.sigmoid(h1) * h3).astype(jnp.bfloat16)
        y = lax.dot_general(g, w2e, nt, preferred_element_type=jnp.float32)
        sl = pl.ds(b * R, R)

        @pl.when(f == 0)
        def _():
            acc_ref[sl, :] = y

        @pl.when(f > 0)
        def _():
            acc_ref[sl, :] = acc_ref[sl, :] + y

        @pl.when(f == F - 1)
        def _():
            ys_ref[...] = acc_ref[sl, :]


def _ffn(blk, xs, w1, w3, w2):
    grid_spec = pltpu.PrefetchScalarGridSpec(
        num_scalar_prefetch=1,
        grid=(F, G_MAX),
        in_specs=[
            pl.BlockSpec((R, D),
                         lambda f, b, s: (jnp.minimum(b, s[NB_SLOT] - 1), 0)),
            pl.BlockSpec((1, DFB, D), lambda f, b, s: (s[b], f, 0)),
            pl.BlockSpec((1, DFB, D), lambda f, b, s: (s[b], f, 0)),
            pl.BlockSpec((1, D, DFB), lambda f, b, s: (s[b], 0, f)),
        ],
        out_specs=pl.BlockSpec((R, D),
                               lambda f, b, s: (jnp.where(f == F - 1, b, 0), 0)),
        scratch_shapes=[pltpu.VMEM((CAP, D), jnp.float32)],
    )
    return pl.pallas_call(
        _ffn_body,
        grid_spec=grid_spec,
        out_shape=jax.ShapeDtypeStruct((CAP, D), jnp.float32),
    )(blk, xs, w1, w3, w2)


def _combine_body(ys_hbm, pos0_hbm, pos1_hbm, cw0_hbm, cw1_hbm, out_hbm,
                  p0v, p1v, w0v, w1v, buf0, buf1):
    cid = lax.axis_index("c")
    sid = lax.axis_index("s")
    wid = sid * 2 + cid
    base = wid * TPW

    pltpu.sync_copy(pos0_hbm.at[pl.ds(base, TPW)], p0v.at[0])
    pltpu.sync_copy(pos1_hbm.at[pl.ds(base, TPW)], p1v.at[0])
    pltpu.sync_copy(cw0_hbm.at[pl.ds(base, TPW)], w0v)
    pltpu.sync_copy(cw1_hbm.at[pl.ds(base, TPW)], w1v)
    pltpu.sync_copy(ys_hbm.at[p0v.at[0]], buf0)
    pltpu.sync_copy(ys_hbm.at[p1v.at[0]], buf1)

    @pl.loop(0, TPW)
    def _(t):
        a = w0v[t]
        c = w1v[t]

        @pl.loop(0, D // 16)
        def _(dd):
            sl = pl.ds(dd * 16, 16)
            buf0[t, sl] = a * buf0[t, sl] + c * buf1[t, sl]

    pltpu.sync_copy(buf0, out_hbm.at[pl.ds(base, TPW)])


def _combine(ys, pos0, pos1, cw0, cw1):
    mesh = plsc.VectorSubcoreMesh(core_axis_name="c", subcore_axis_name="s")
    return pl.kernel(
        _combine_body,
        out_type=jax.ShapeDtypeStruct((T, D), jnp.float32),
        mesh=mesh,
        scratch_types=[
            pltpu.VMEM((1, TPW), jnp.int32),
            pltpu.VMEM((1, TPW), jnp.int32),
            pltpu.VMEM((TPW,), jnp.float32),
            pltpu.VMEM((TPW,), jnp.float32),
            pltpu.VMEM((TPW, D), jnp.float32),
            pltpu.VMEM((TPW, D), jnp.float32),
        ],
    )(ys, pos0, pos1, cw0, cw1)


def kernel(hidden_states, W_gate, w1, w3, w2):
    orig_shape = hidden_states.shape
    x = hidden_states.reshape(-1, D)
    wg_pad = jnp.pad(W_gate, ((0, 0), (0, EPAD - E)))

    route = _router(x, wg_pad)
    i0 = route[:, 0].astype(jnp.int32)
    i1 = route[:, 1].astype(jnp.int32)
    cw0 = route[:, 2]
    cw1 = route[:, 3]

    xs, pos0, pos1, blk = _dispatch(x, i0, i1)
    ys = _ffn(blk, xs, w1, w3, w2)
    out = _combine(ys, pos0, pos1, cw0, cw1)
    return out.reshape(orig_shape)


# speedup vs baseline: 1.3576x; 1.3576x over previous
import dataclasses

import jax
import jax.numpy as jnp
from jax import lax
from jax.experimental import pallas as pl
from jax.experimental.pallas import tpu as pltpu
from jax.experimental.pallas import tpu_sc as plsc

E = 8
TOPK = 2
D = 768
DFF = 2688
T = 2048

R = 256
CAP = T * TOPK + E * R
G_MAX = CAP // R
F = 3
DFB = DFF // F
EPAD = 128

NW = 32
TPW = T // NW
NB_SLOT = G_MAX


def _sc_compiler_params():
    cp = pltpu.CompilerParams()
    if "needs_layout_passes" in pltpu.CompilerParams.__dataclass_fields__:
        cp = dataclasses.replace(cp, needs_layout_passes=False)
    return cp


def _lane16():
    return lax.iota(jnp.int32, 16)


def _vlane(v, e):
    return jnp.sum(jnp.where(_lane16() == e, v, jnp.zeros_like(v)))


def _router_body(x_ref, wg_ref, o_ref):
    lg = jnp.dot(x_ref[...], wg_ref[...], preferred_element_type=jnp.float32)
    lane = lax.broadcasted_iota(jnp.int32, lg.shape, 1)
    valid = lane < E
    lgm = jnp.where(valid, lg, jnp.float32(-1e30))
    m = jnp.max(lgm, axis=1, keepdims=True)
    ex = jnp.where(valid, jnp.exp(lgm - m), 0.0)
    p = ex / jnp.sum(ex, axis=1, keepdims=True)
    m1 = jnp.max(p, axis=1, keepdims=True)
    i1 = jnp.min(jnp.where(p >= m1, lane, EPAD), axis=1, keepdims=True)
    p2 = jnp.where(lane == i1, jnp.float32(-1.0), p)
    m2 = jnp.max(p2, axis=1, keepdims=True)
    i2 = jnp.min(jnp.where(p2 >= m2, lane, EPAD), axis=1, keepdims=True)
    s = m1 + m2
    o_ref[...] = jnp.where(
        lane == 0, i1.astype(jnp.float32),
        jnp.where(lane == 1, i2.astype(jnp.float32),
                  jnp.where(lane == 2, m1 / s,
                            jnp.where(lane == 3, m2 / s, 0.0))))


def _router(x, wg_pad):
    return pl.pallas_call(
        _router_body,
        grid=(T // R,),
        in_specs=[
            pl.BlockSpec((R, D), lambda i: (i, 0)),
            pl.BlockSpec((D, EPAD), lambda i: (0, 0)),
        ],
        out_specs=pl.BlockSpec((R, EPAD), lambda i: (i, 0)),
        out_shape=jax.ShapeDtypeStruct((T, EPAD), jnp.float32),
    )(x, wg_pad)


def _dispatch_body(x_hbm, i0_hbm, i1_hbm,
                   xs_hbm, pos0_hbm, pos1_hbm, blk_hbm,
                   ivm0, ivm1, xrows, p0v, p1v, blkv):
    cid = lax.axis_index("c")
    sid = lax.axis_index("s")
    wid = sid * 2 + cid
    mybase = wid * TPW
    zeros16 = jnp.zeros((16,), jnp.int32)
    lanes = _lane16()

    pltpu.sync_copy(i0_hbm, ivm0)
    pltpu.sync_copy(i1_hbm, ivm1)
    pltpu.sync_copy(x_hbm.at[pl.ds(mybase, TPW)], xrows)

    def chunk_step(w, carry):
        totals, prefix = carry
        wv = jnp.full((16,), w, jnp.int32)
        is_prev = wv < jnp.full((16,), wid, jnp.int32)
        base = w * TPW
        for r in range(TPW // 16):
            for src in (ivm0, ivm1):
                v = src[pl.ds(base + r * 16, 16)]
                for e in range(E):
                    cnt = plsc.all_reduce_population_count(v == e)
                    delta = jnp.where(lanes == e, cnt, zeros16)
                    totals = totals + delta
                    prefix = prefix + jnp.where(is_prev, delta, zeros16)
        return totals, prefix

    totals, prefix = lax.fori_loop(0, NW, chunk_step, (zeros16, zeros16))

    padded = ((totals + (R - 1)) // R) * R
    incl = plsc.cumsum(padded)
    seg_start = incl - padded
    nb = _vlane(incl, E - 1) // R

    running = seg_start + prefix
    for r in range(TPW // 16):
        for src, dst in ((ivm0, p0v), (ivm1, p1v)):
            v = src[pl.ds(mybase + r * 16, 16)]
            posv = zeros16
            for e in range(E):
                m = v == e
                ones = jnp.where(m, jnp.ones((16,), jnp.int32), zeros16)
                rank = plsc.cumsum(ones) - 1
                b_e = _vlane(running, e)
                posv = jnp.where(m, b_e + rank, posv)
                running = running + jnp.where(lanes == e, jnp.sum(ones), 0)
            dst[0, pl.ds(r * 16, 16)] = posv

    pltpu.sync_copy(p0v.at[0], pos0_hbm.at[pl.ds(mybase, TPW)])
    pltpu.sync_copy(p1v.at[0], pos1_hbm.at[pl.ds(mybase, TPW)])
    pltpu.sync_copy(xrows, xs_hbm.at[p0v.at[0]])
    pltpu.sync_copy(xrows, xs_hbm.at[p1v.at[0]])

    @pl.when(wid == 0)
    def _():
        for half in range(2):
            bidx = lanes + half * 16
            beff = jnp.minimum(bidx, nb - 1)
            posb = beff * R
            ev = zeros16
            for e in range(E):
                s_e = _vlane(seg_start, e)
                p_e = _vlane(padded, e)
                m = (posb >= s_e) & (posb < s_e + p_e)
                ev = jnp.where(m, jnp.full((16,), e, jnp.int32), ev)
            if half == 1:
                ev = jnp.where(lanes == NB_SLOT - 16,
                               jnp.full((16,), nb, jnp.int32), ev)
            blkv[0, pl.ds(half * 16, 16)] = ev
        pltpu.sync_copy(blkv.at[0], blk_hbm)


def _dispatch(x, i0, i1):
    mesh = plsc.VectorSubcoreMesh(core_axis_name="c", subcore_axis_name="s")
    return pl.kernel(
        _dispatch_body,
        out_type=(
            jax.ShapeDtypeStruct((CAP, D), jnp.float32),
            jax.ShapeDtypeStruct((T,), jnp.int32),
            jax.ShapeDtypeStruct((T,), jnp.int32),
            jax.ShapeDtypeStruct((32,), jnp.int32),
        ),
        mesh=mesh,
        scratch_types=[
            pltpu.VMEM((T,), jnp.int32),
            pltpu.VMEM((T,), jnp.int32),
            pltpu.VMEM((TPW, D), jnp.float32),
            pltpu.VMEM((1, TPW), jnp.int32),
            pltpu.VMEM((1, TPW), jnp.int32),
            pltpu.VMEM((1, 32), jnp.int32),
        ],
        compiler_params=_sc_compiler_params(),
    )(x, i0, i1)


def _ffn_body(s_ref, xs_ref, w1_ref, w3_ref, w2_ref, ys_ref, acc_ref):
    f = pl.program_id(0)
    b = pl.program_id(1)

    @pl.when(b < s_ref[NB_SLOT])
    def _():
        xb = xs_ref[...].astype(jnp.bfloat16)
        w1e = w1_ref[0].astype(jnp.bfloat16)
        w3e = w3_ref[0].astype(jnp.bfloat16)
        w2e = w2_ref[0].astype(jnp.bfloat16)
        nt = (((1,), (1,)), ((), ()))
        h1 = lax.dot_general(xb, w1e, nt, preferred_element_type=jnp.float32)
        h3 = lax.dot_general(xb, w3e, nt, preferred_element_type=jnp.float32)
        g = (h1 * jax.nn.sigmoid(h1) * h3).astype(jnp.bfloat16)
        y = lax.dot_general(g, w2e, nt, preferred_element_type=jnp.float32)
        sl = pl.ds(b * R, R)

        @pl.when(f == 0)
        def _():
            acc_ref[sl, :] = y

        @pl.when(f > 0)
        def _():
            acc_ref[sl, :] = acc_ref[sl, :] + y

        @pl.when(f == F - 1)
        def _():
            ys_ref[...] = acc_ref[sl, :]


def _ffn(blk, xs, w1, w3, w2):
    grid_spec = pltpu.PrefetchScalarGridSpec(
        num_scalar_prefetch=1,
        grid=(F, G_MAX),
        in_specs=[
            pl.BlockSpec((R, D),
                         lambda f, b, s: (jnp.minimum(b, s[NB_SLOT] - 1), 0)),
            pl.BlockSpec((1, DFB, D), lambda f, b, s: (s[b], f, 0)),
            pl.BlockSpec((1, DFB, D), lambda f, b, s: (s[b], f, 0)),
            pl.BlockSpec((1, D, DFB), lambda f, b, s: (s[b], 0, f)),
        ],
        out_specs=pl.BlockSpec((R, D),
                               lambda f, b, s: (jnp.where(f == F - 1, b, 0), 0)),
        scratch_shapes=[pltpu.VMEM((CAP, D), jnp.float32)],
    )
    return pl.pallas_call(
        _ffn_body,
        grid_spec=grid_spec,
        out_shape=jax.ShapeDtypeStruct((CAP, D), jnp.float32),
    )(blk, xs, w1, w3, w2)


def _combine_body(ys_hbm, pos0_hbm, pos1_hbm, cw0_hbm, cw1_hbm, out_hbm,
                  p0v, p1v, w0v, w1v, buf0, buf1):
    cid = lax.axis_index("c")
    sid = lax.axis_index("s")
    wid = sid * 2 + cid
    base = wid * TPW

    pltpu.sync_copy(pos0_hbm.at[pl.ds(base, TPW)], p0v.at[0])
    pltpu.sync_copy(pos1_hbm.at[pl.ds(base, TPW)], p1v.at[0])
    pltpu.sync_copy(cw0_hbm.at[pl.ds(base, TPW)], w0v)
    pltpu.sync_copy(cw1_hbm.at[pl.ds(base, TPW)], w1v)
    pltpu.sync_copy(ys_hbm.at[p0v.at[0]], buf0)
    pltpu.sync_copy(ys_hbm.at[p1v.at[0]], buf1)

    for tq in range(TPW // 16):
        wa = w0v[pl.ds(tq * 16, 16)]
        wc = w1v[pl.ds(tq * 16, 16)]
        for k in range(16):
            t = tq * 16 + k

            @pl.loop(0, D // 16)
            def _(dd, t=t, a=wa[k], c=wc[k]):
                sl = pl.ds(dd * 16, 16)
                buf0[t, sl] = a * buf0[t, sl] + c * buf1[t, sl]

    pltpu.sync_copy(buf0, out_hbm.at[pl.ds(base, TPW)])


def _combine(ys, pos0, pos1, cw0, cw1):
    mesh = plsc.VectorSubcoreMesh(core_axis_name="c", subcore_axis_name="s")
    return pl.kernel(
        _combine_body,
        out_type=jax.ShapeDtypeStruct((T, D), jnp.float32),
        mesh=mesh,
        scratch_types=[
            pltpu.VMEM((1, TPW), jnp.int32),
            pltpu.VMEM((1, TPW), jnp.int32),
            pltpu.VMEM((TPW,), jnp.float32),
            pltpu.VMEM((TPW,), jnp.float32),
            pltpu.VMEM((TPW, D), jnp.float32),
            pltpu.VMEM((TPW, D), jnp.float32),
        ],
        compiler_params=_sc_compiler_params(),
    )(ys, pos0, pos1, cw0, cw1)


def kernel(hidden_states, W_gate, w1, w3, w2):
    orig_shape = hidden_states.shape
    x = hidden_states.reshape(-1, D)
    wg_pad = jnp.pad(W_gate, ((0, 0), (0, EPAD - E)))

    route = _router(x, wg_pad)
    i0 = route[:, 0].astype(jnp.int32)
    i1 = route[:, 1].astype(jnp.int32)
    cw0 = route[:, 2]
    cw1 = route[:, 3]

    xs, pos0, pos1, blk = _dispatch(x, i0, i1)
    ys = _ffn(blk, xs, w1, w3, w2)
    out = _combine(ys, pos0, pos1, cw0, cw1)
    return out.reshape(orig_shape)


# speedup vs baseline: 1.3851x; 1.0202x over previous
import dataclasses

import jax
import jax.numpy as jnp
from jax import lax
from jax.experimental import pallas as pl
from jax.experimental.pallas import tpu as pltpu
from jax.experimental.pallas import tpu_sc as plsc

E = 8
TOPK = 2
D = 768
DFF = 2688
T = 2048

R = 256
CAP = T * TOPK + E * R
G_MAX = CAP // R
F = 3
DFB = DFF // F
EPAD = 128

NW = 32
TPW = T // NW
NB_SLOT = G_MAX


def _sc_compiler_params():
    cp = pltpu.CompilerParams()
    if "needs_layout_passes" in pltpu.CompilerParams.__dataclass_fields__:
        cp = dataclasses.replace(cp, needs_layout_passes=False)
    return cp


def _lane16():
    return lax.iota(jnp.int32, 16)


def _vlane(v, e):
    return jnp.sum(jnp.where(_lane16() == e, v, jnp.zeros_like(v)))


def _router_body(x_ref, wg_ref, o_ref):
    lg = jnp.dot(x_ref[...], wg_ref[...], preferred_element_type=jnp.float32)
    lane = lax.broadcasted_iota(jnp.int32, lg.shape, 1)
    valid = lane < E
    lgm = jnp.where(valid, lg, jnp.float32(-1e30))
    m = jnp.max(lgm, axis=1, keepdims=True)
    ex = jnp.where(valid, jnp.exp(lgm - m), 0.0)
    p = ex / jnp.sum(ex, axis=1, keepdims=True)
    m1 = jnp.max(p, axis=1, keepdims=True)
    i1 = jnp.min(jnp.where(p >= m1, lane, EPAD), axis=1, keepdims=True)
    p2 = jnp.where(lane == i1, jnp.float32(-1.0), p)
    m2 = jnp.max(p2, axis=1, keepdims=True)
    i2 = jnp.min(jnp.where(p2 >= m2, lane, EPAD), axis=1, keepdims=True)
    s = m1 + m2
    o_ref[...] = jnp.where(
        lane == 0, i1.astype(jnp.float32),
        jnp.where(lane == 1, i2.astype(jnp.float32),
                  jnp.where(lane == 2, m1 / s,
                            jnp.where(lane == 3, m2 / s, 0.0))))


def _router(x, wg_pad):
    return pl.pallas_call(
        _router_body,
        grid=(T // R,),
        in_specs=[
            pl.BlockSpec((R, D), lambda i: (i, 0)),
            pl.BlockSpec((D, EPAD), lambda i: (0, 0)),
        ],
        out_specs=pl.BlockSpec((R, EPAD), lambda i: (i, 0)),
        out_shape=jax.ShapeDtypeStruct((T, EPAD), jnp.float32),
    )(x, wg_pad)


def _dispatch_body(x_hbm, i0_hbm, i1_hbm,
                   xs_hbm, pos0_hbm, pos1_hbm, blk_hbm,
                   ivm0, ivm1, xrows, p0v, p1v, blkv):
    cid = lax.axis_index("c")
    sid = lax.axis_index("s")
    wid = sid * 2 + cid
    mybase = wid * TPW
    zeros16 = jnp.zeros((16,), jnp.int32)
    lanes = _lane16()

    pltpu.sync_copy(i0_hbm, ivm0)
    pltpu.sync_copy(i1_hbm, ivm1)
    pltpu.sync_copy(x_hbm.at[pl.ds(mybase, TPW)], xrows)

    def chunk_step(w, carry):
        totals, prefix = carry
        wv = jnp.full((16,), w, jnp.int32)
        is_prev = wv < jnp.full((16,), wid, jnp.int32)
        base = w * TPW
        for r in range(TPW // 16):
            for src in (ivm0, ivm1):
                v = src[pl.ds(base + r * 16, 16)]
                for e in range(E):
                    cnt = plsc.all_reduce_population_count(v == e)
                    delta = jnp.where(lanes == e, cnt, zeros16)
                    totals = totals + delta
                    prefix = prefix + jnp.where(is_prev, delta, zeros16)
        return totals, prefix

    totals, prefix = lax.fori_loop(0, NW, chunk_step, (zeros16, zeros16))

    padded = ((totals + (R - 1)) // R) * R
    incl = plsc.cumsum(padded)
    seg_start = incl - padded
    nb = _vlane(incl, E - 1) // R

    running = seg_start + prefix
    for r in range(TPW // 16):
        for src, dst in ((ivm0, p0v), (ivm1, p1v)):
            v = src[pl.ds(mybase + r * 16, 16)]
            posv = zeros16
            for e in range(E):
                m = v == e
                ones = jnp.where(m, jnp.ones((16,), jnp.int32), zeros16)
                rank = plsc.cumsum(ones) - 1
                b_e = _vlane(running, e)
                posv = jnp.where(m, b_e + rank, posv)
                running = running + jnp.where(lanes == e, jnp.sum(ones), 0)
            dst[0, pl.ds(r * 16, 16)] = posv

    pltpu.sync_copy(p0v.at[0], pos0_hbm.at[pl.ds(mybase, TPW)])
    pltpu.sync_copy(p1v.at[0], pos1_hbm.at[pl.ds(mybase, TPW)])
    pltpu.sync_copy(xrows, xs_hbm.at[p0v.at[0]])
    pltpu.sync_copy(xrows, xs_hbm.at[p1v.at[0]])

    @pl.when(wid == 0)
    def _():
        for half in range(2):
            bidx = lanes + half * 16
            beff = jnp.minimum(bidx, nb - 1)
            posb = beff * R
            ev = zeros16
            for e in range(E):
                s_e = _vlane(seg_start, e)
                p_e = _vlane(padded, e)
                m = (posb >= s_e) & (posb < s_e + p_e)
                ev = jnp.where(m, jnp.full((16,), e, jnp.int32), ev)
            if half == 1:
                ev = jnp.where(lanes == NB_SLOT - 16,
                               jnp.full((16,), nb, jnp.int32), ev)
            blkv[0, pl.ds(half * 16, 16)] = ev
        pltpu.sync_copy(blkv.at[0], blk_hbm)


def _dispatch(x, i0, i1):
    mesh = plsc.VectorSubcoreMesh(core_axis_name="c", subcore_axis_name="s")
    return pl.kernel(
        _dispatch_body,
        out_type=(
            jax.ShapeDtypeStruct((CAP, D), jnp.float32),
            jax.ShapeDtypeStruct((T,), jnp.int32),
            jax.ShapeDtypeStruct((T,), jnp.int32),
            jax.ShapeDtypeStruct((32,), jnp.int32),
        ),
        mesh=mesh,
        scratch_types=[
            pltpu.VMEM((T,), jnp.int32),
            pltpu.VMEM((T,), jnp.int32),
            pltpu.VMEM((TPW, D), jnp.float32),
            pltpu.VMEM((1, TPW), jnp.int32),
            pltpu.VMEM((1, TPW), jnp.int32),
            pltpu.VMEM((1, 32), jnp.int32),
        ],
        compiler_params=_sc_compiler_params(),
    )(x, i0, i1)


def _ffn_body(s_ref, xs_ref, w1_ref, w3_ref, w2_ref, ys_ref, acc_ref,
              w1c_ref, w3c_ref, w2c_ref):
    f = pl.program_id(0)
    b = pl.program_id(1)

    @pl.when(b < s_ref[NB_SLOT])
    def _():
        bm1 = jnp.maximum(b - 1, 0)

        @pl.when((b == 0) | (s_ref[b] != s_ref[bm1]))
        def _():
            w1c_ref[...] = w1_ref[0].astype(jnp.bfloat16)
            w3c_ref[...] = w3_ref[0].astype(jnp.bfloat16)
            w2c_ref[...] = w2_ref[0].astype(jnp.bfloat16)

        xb = xs_ref[...].astype(jnp.bfloat16)
        w1e = w1c_ref[...]
        w3e = w3c_ref[...]
        w2e = w2c_ref[...]
        nt = (((1,), (1,)), ((), ()))
        h1 = lax.dot_general(xb, w1e, nt, preferred_element_type=jnp.float32)
        h3 = lax.dot_general(xb, w3e, nt, preferred_element_type=jnp.float32)
        g = (h1 * jax.nn.sigmoid(h1) * h3).astype(jnp.bfloat16)
        y = lax.dot_general(g, w2e, nt, preferred_element_type=jnp.float32)
        sl = pl.ds(b * R, R)

        @pl.when(f == 0)
        def _():
            acc_ref[sl, :] = y

        @pl.when(f > 0)
        def _():
            acc_ref[sl, :] = acc_ref[sl, :] + y

        @pl.when(f == F - 1)
        def _():
            ys_ref[...] = acc_ref[sl, :]


def _ffn(blk, xs, w1, w3, w2):
    grid_spec = pltpu.PrefetchScalarGridSpec(
        num_scalar_prefetch=1,
        grid=(F, G_MAX),
        in_specs=[
            pl.BlockSpec((R, D),
                         lambda f, b, s: (jnp.minimum(b, s[NB_SLOT] - 1), 0)),
            pl.BlockSpec((1, DFB, D), lambda f, b, s: (s[b], f, 0)),
            pl.BlockSpec((1, DFB, D), lambda f, b, s: (s[b], f, 0)),
            pl.BlockSpec((1, D, DFB), lambda f, b, s: (s[b], 0, f)),
        ],
        out_specs=pl.BlockSpec((R, D),
                               lambda f, b, s: (jnp.where(f == F - 1, b, 0), 0)),
        scratch_shapes=[
            pltpu.VMEM((CAP, D), jnp.float32),
            pltpu.VMEM((DFB, D), jnp.bfloat16),
            pltpu.VMEM((DFB, D), jnp.bfloat16),
            pltpu.VMEM((D, DFB), jnp.bfloat16),
        ],
    )
    return pl.pallas_call(
        _ffn_body,
        grid_spec=grid_spec,
        out_shape=jax.ShapeDtypeStruct((CAP, D), jnp.float32),
    )(blk, xs, w1, w3, w2)


def _combine_body(ys_hbm, pos0_hbm, pos1_hbm, cw0_hbm, cw1_hbm, out_hbm,
                  p0v, p1v, w0v, w1v, buf0, buf1, sem0, sem1, sem2, sem3):
    cid = lax.axis_index("c")
    sid = lax.axis_index("s")
    wid = sid * 2 + cid
    base = wid * TPW

    cp0 = pltpu.async_copy(pos0_hbm.at[pl.ds(base, TPW)], p0v.at[0], sem0)
    cp1 = pltpu.async_copy(pos1_hbm.at[pl.ds(base, TPW)], p1v.at[0], sem1)
    cwa = pltpu.async_copy(cw0_hbm.at[pl.ds(base, TPW)], w0v, sem2)
    cwc = pltpu.async_copy(cw1_hbm.at[pl.ds(base, TPW)], w1v, sem3)
    cp0.wait()
    g0 = pltpu.async_copy(ys_hbm.at[p0v.at[0]], buf0, sem0)
    cp1.wait()
    g1 = pltpu.async_copy(ys_hbm.at[p1v.at[0]], buf1, sem1)
    cwa.wait()
    cwc.wait()
    g0.wait()
    g1.wait()

    for tq in range(TPW // 16):
        wa = w0v[pl.ds(tq * 16, 16)]
        wc = w1v[pl.ds(tq * 16, 16)]
        for k in range(16):
            t = tq * 16 + k

            @plsc.parallel_loop(0, D // 16, unroll=4)
            def _(dd, t=t, a=wa[k], c=wc[k]):
                sl = pl.ds(dd * 16, 16)
                buf0[t, sl] = a * buf0[t, sl] + c * buf1[t, sl]

    pltpu.sync_copy(buf0, out_hbm.at[pl.ds(base, TPW)])


def _combine(ys, pos0, pos1, cw0, cw1):
    mesh = plsc.VectorSubcoreMesh(core_axis_name="c", subcore_axis_name="s")
    return pl.kernel(
        _combine_body,
        out_type=jax.ShapeDtypeStruct((T, D), jnp.float32),
        mesh=mesh,
        scratch_types=[
            pltpu.VMEM((1, TPW), jnp.int32),
            pltpu.VMEM((1, TPW), jnp.int32),
            pltpu.VMEM((TPW,), jnp.float32),
            pltpu.VMEM((TPW,), jnp.float32),
            pltpu.VMEM((TPW, D), jnp.float32),
            pltpu.VMEM((TPW, D), jnp.float32),
            pltpu.SemaphoreType.DMA,
            pltpu.SemaphoreType.DMA,
            pltpu.SemaphoreType.DMA,
            pltpu.SemaphoreType.DMA,
        ],
        compiler_params=_sc_compiler_params(),
    )(ys, pos0, pos1, cw0, cw1)


def kernel(hidden_states, W_gate, w1, w3, w2):
    orig_shape = hidden_states.shape
    x = hidden_states.reshape(-1, D)
    wg_pad = jnp.pad(W_gate, ((0, 0), (0, EPAD - E)))

    route = _router(x, wg_pad)
    i0 = route[:, 0].astype(jnp.int32)
    i1 = route[:, 1].astype(jnp.int32)
    cw0 = route[:, 2]
    cw1 = route[:, 3]

    xs, pos0, pos1, blk = _dispatch(x, i0, i1)
    ys = _ffn(blk, xs, w1, w3, w2)
    out = _combine(ys, pos0, pos1, cw0, cw1)
    return out.reshape(orig_shape)


# speedup vs baseline: 1.6009x; 1.1558x over previous
import dataclasses

import jax
import jax.numpy as jnp
from jax import lax
from jax.experimental import pallas as pl
from jax.experimental.pallas import tpu as pltpu
from jax.experimental.pallas import tpu_sc as plsc

E = 8
TOPK = 2
D = 768
DFF = 2688
T = 2048

R = 512
CAP = T * TOPK + E * R
G_MAX = CAP // R
F = 3
DFB = DFF // F
EPAD = 128

NW = 32
TPW = T // NW
NB_SLOT = G_MAX


def _sc_compiler_params():
    cp = pltpu.CompilerParams()
    if "needs_layout_passes" in pltpu.CompilerParams.__dataclass_fields__:
        cp = dataclasses.replace(cp, needs_layout_passes=False)
    return cp


def _lane16():
    return lax.iota(jnp.int32, 16)


def _vlane(v, e):
    return jnp.sum(jnp.where(_lane16() == e, v, jnp.zeros_like(v)))


def _router_body(x_ref, wg_ref, o_ref):
    lg = jnp.dot(x_ref[...], wg_ref[...], preferred_element_type=jnp.float32)
    lane = lax.broadcasted_iota(jnp.int32, lg.shape, 1)
    valid = lane < E
    lgm = jnp.where(valid, lg, jnp.float32(-1e30))
    m = jnp.max(lgm, axis=1, keepdims=True)
    ex = jnp.where(valid, jnp.exp(lgm - m), 0.0)
    p = ex / jnp.sum(ex, axis=1, keepdims=True)
    m1 = jnp.max(p, axis=1, keepdims=True)
    i1 = jnp.min(jnp.where(p >= m1, lane, EPAD), axis=1, keepdims=True)
    p2 = jnp.where(lane == i1, jnp.float32(-1.0), p)
    m2 = jnp.max(p2, axis=1, keepdims=True)
    i2 = jnp.min(jnp.where(p2 >= m2, lane, EPAD), axis=1, keepdims=True)
    s = m1 + m2
    o_ref[...] = jnp.where(
        lane == 0, i1.astype(jnp.float32),
        jnp.where(lane == 1, i2.astype(jnp.float32),
                  jnp.where(lane == 2, m1 / s,
                            jnp.where(lane == 3, m2 / s, 0.0))))


def _router(x, wg_pad):
    return pl.pallas_call(
        _router_body,
        grid=(T // R,),
        in_specs=[
            pl.BlockSpec((R, D), lambda i: (i, 0)),
            pl.BlockSpec((D, EPAD), lambda i: (0, 0)),
        ],
        out_specs=pl.BlockSpec((R, EPAD), lambda i: (i, 0)),
        out_shape=jax.ShapeDtypeStruct((T, EPAD), jnp.float32),
    )(x, wg_pad)


def _dispatch_body(x_hbm, i0_hbm, i1_hbm,
                   xs_hbm, pos0_hbm, pos1_hbm, blk_hbm,
                   ivm0, ivm1, xrows, p0v, p1v, blkv):
    cid = lax.axis_index("c")
    sid = lax.axis_index("s")
    wid = sid * 2 + cid
    mybase = wid * TPW
    zeros16 = jnp.zeros((16,), jnp.int32)
    lanes = _lane16()

    pltpu.sync_copy(i0_hbm, ivm0)
    pltpu.sync_copy(i1_hbm, ivm1)
    pltpu.sync_copy(x_hbm.at[pl.ds(mybase, TPW)], xrows)

    def chunk_step(w, carry):
        totals, prefix = carry
        wv = jnp.full((16,), w, jnp.int32)
        is_prev = wv < jnp.full((16,), wid, jnp.int32)
        base = w * TPW
        for r in range(TPW // 16):
            for src in (ivm0, ivm1):
                v = src[pl.ds(base + r * 16, 16)]
                for e in range(E):
                    cnt = plsc.all_reduce_population_count(v == e)
                    delta = jnp.where(lanes == e, cnt, zeros16)
                    totals = totals + delta
                    prefix = prefix + jnp.where(is_prev, delta, zeros16)
        return totals, prefix

    totals, prefix = lax.fori_loop(0, NW, chunk_step, (zeros16, zeros16))

    padded = ((totals + (R - 1)) // R) * R
    incl = plsc.cumsum(padded)
    seg_start = incl - padded
    nb = _vlane(incl, E - 1) // R

    running = seg_start + prefix
    for r in range(TPW // 16):
        for src, dst in ((ivm0, p0v), (ivm1, p1v)):
            v = src[pl.ds(mybase + r * 16, 16)]
            posv = zeros16
            for e in range(E):
                m = v == e
                ones = jnp.where(m, jnp.ones((16,), jnp.int32), zeros16)
                rank = plsc.cumsum(ones) - 1
                b_e = _vlane(running, e)
                posv = jnp.where(m, b_e + rank, posv)
                running = running + jnp.where(lanes == e, jnp.sum(ones), 0)
            dst[0, pl.ds(r * 16, 16)] = posv

    pltpu.sync_copy(p0v.at[0], pos0_hbm.at[pl.ds(mybase, TPW)])
    pltpu.sync_copy(p1v.at[0], pos1_hbm.at[pl.ds(mybase, TPW)])
    pltpu.sync_copy(xrows, xs_hbm.at[p0v.at[0]])
    pltpu.sync_copy(xrows, xs_hbm.at[p1v.at[0]])

    @pl.when(wid == 0)
    def _():
        for half in range(2):
            bidx = lanes + half * 16
            beff = jnp.minimum(bidx, nb - 1)
            posb = beff * R
            ev = zeros16
            for e in range(E):
                s_e = _vlane(seg_start, e)
                p_e = _vlane(padded, e)
                m = (posb >= s_e) & (posb < s_e + p_e)
                ev = jnp.where(m, jnp.full((16,), e, jnp.int32), ev)
            if half == 1:
                ev = jnp.where(lanes == NB_SLOT - 16,
                               jnp.full((16,), nb, jnp.int32), ev)
            blkv[0, pl.ds(half * 16, 16)] = ev
        pltpu.sync_copy(blkv.at[0], blk_hbm)


def _dispatch(x, i0, i1):
    mesh = plsc.VectorSubcoreMesh(core_axis_name="c", subcore_axis_name="s")
    return pl.kernel(
        _dispatch_body,
        out_type=(
            jax.ShapeDtypeStruct((CAP, D), jnp.float32),
            jax.ShapeDtypeStruct((T,), jnp.int32),
            jax.ShapeDtypeStruct((T,), jnp.int32),
            jax.ShapeDtypeStruct((32,), jnp.int32),
        ),
        mesh=mesh,
        scratch_types=[
            pltpu.VMEM((T,), jnp.int32),
            pltpu.VMEM((T,), jnp.int32),
            pltpu.VMEM((TPW, D), jnp.float32),
            pltpu.VMEM((1, TPW), jnp.int32),
            pltpu.VMEM((1, TPW), jnp.int32),
            pltpu.VMEM((1, 32), jnp.int32),
        ],
        compiler_params=_sc_compiler_params(),
    )(x, i0, i1)


def _ffn_body(s_ref, xs_ref, w1_ref, w3_ref, w2_ref, ys_ref, acc_ref,
              w1c_ref, w3c_ref, w2c_ref):
    f = pl.program_id(0)
    b = pl.program_id(1)

    @pl.when(b < s_ref[NB_SLOT])
    def _():
        bm1 = jnp.maximum(b - 1, 0)

        @pl.when((b == 0) | (s_ref[b] != s_ref[bm1]))
        def _():
            w1c_ref[...] = w1_ref[0].astype(jnp.bfloat16)
            w3c_ref[...] = w3_ref[0].astype(jnp.bfloat16)
            w2c_ref[...] = w2_ref[0].astype(jnp.bfloat16)

        xb = xs_ref[...].astype(jnp.bfloat16)
        w1e = w1c_ref[...]
        w3e = w3c_ref[...]
        w2e = w2c_ref[...]
        nt = (((1,), (1,)), ((), ()))
        h1 = lax.dot_general(xb, w1e, nt,
                             preferred_element_type=jnp.float32).astype(jnp.bfloat16)
        h3 = lax.dot_general(xb, w3e, nt,
                             preferred_element_type=jnp.float32).astype(jnp.bfloat16)
        g = h1 * jax.nn.sigmoid(h1) * h3
        y = lax.dot_general(g, w2e, nt, preferred_element_type=jnp.float32)
        sl = pl.ds(b * R, R)

        @pl.when(f == 0)
        def _():
            acc_ref[sl, :] = y.astype(jnp.bfloat16)

        @pl.when(f > 0)
        def _():
            acc_ref[sl, :] = acc_ref[sl, :] + y.astype(jnp.bfloat16)

        @pl.when(f == F - 1)
        def _():
            ys_ref[...] = (acc_ref[sl, :]).astype(jnp.float32)


def _ffn(blk, xs, w1, w3, w2):
    grid_spec = pltpu.PrefetchScalarGridSpec(
        num_scalar_prefetch=1,
        grid=(F, G_MAX),
        in_specs=[
            pl.BlockSpec((R, D),
                         lambda f, b, s: (jnp.minimum(b, s[NB_SLOT] - 1), 0)),
            pl.BlockSpec((1, DFB, D), lambda f, b, s: (s[b], f, 0)),
            pl.BlockSpec((1, DFB, D), lambda f, b, s: (s[b], f, 0)),
            pl.BlockSpec((1, D, DFB), lambda f, b, s: (s[b], 0, f)),
        ],
        out_specs=pl.BlockSpec((R, D),
                               lambda f, b, s: (jnp.where(f == F - 1, b, 0), 0)),
        scratch_shapes=[
            pltpu.VMEM((CAP, D), jnp.bfloat16),
            pltpu.VMEM((DFB, D), jnp.bfloat16),
            pltpu.VMEM((DFB, D), jnp.bfloat16),
            pltpu.VMEM((D, DFB), jnp.bfloat16),
        ],
    )
    return pl.pallas_call(
        _ffn_body,
        grid_spec=grid_spec,
        out_shape=jax.ShapeDtypeStruct((CAP, D), jnp.float32),
    )(blk, xs, w1, w3, w2)


def _combine_body(ys_hbm, pos0_hbm, pos1_hbm, cw0_hbm, cw1_hbm, out_hbm,
                  p0v, p1v, w0v, w1v, buf0, buf1, sem0, sem1, sem2, sem3):
    cid = lax.axis_index("c")
    sid = lax.axis_index("s")
    wid = sid * 2 + cid
    base = wid * TPW

    cp0 = pltpu.async_copy(pos0_hbm.at[pl.ds(base, TPW)], p0v.at[0], sem0)
    cp1 = pltpu.async_copy(pos1_hbm.at[pl.ds(base, TPW)], p1v.at[0], sem1)
    cwa = pltpu.async_copy(cw0_hbm.at[pl.ds(base, TPW)], w0v, sem2)
    cwc = pltpu.async_copy(cw1_hbm.at[pl.ds(base, TPW)], w1v, sem3)
    cp0.wait()
    g0 = pltpu.async_copy(ys_hbm.at[p0v.at[0]], buf0, sem0)
    cp1.wait()
    g1 = pltpu.async_copy(ys_hbm.at[p1v.at[0]], buf1, sem1)
    cwa.wait()
    cwc.wait()
    g0.wait()
    g1.wait()

    for tq in range(TPW // 16):
        wa = w0v[pl.ds(tq * 16, 16)]
        wc = w1v[pl.ds(tq * 16, 16)]
        for k in range(16):
            t = tq * 16 + k

            @plsc.parallel_loop(0, D // 16, unroll=4)
            def _(dd, t=t, a=wa[k], c=wc[k]):
                sl = pl.ds(dd * 16, 16)
                buf0[t, sl] = a * buf0[t, sl] + c * buf1[t, sl]

    pltpu.sync_copy(buf0, out_hbm.at[pl.ds(base, TPW)])


def _combine(ys, pos0, pos1, cw0, cw1):
    mesh = plsc.VectorSubcoreMesh(core_axis_name="c", subcore_axis_name="s")
    return pl.kernel(
        _combine_body,
        out_type=jax.ShapeDtypeStruct((T, D), jnp.float32),
        mesh=mesh,
        scratch_types=[
            pltpu.VMEM((1, TPW), jnp.int32),
            pltpu.VMEM((1, TPW), jnp.int32),
            pltpu.VMEM((TPW,), jnp.float32),
            pltpu.VMEM((TPW,), jnp.float32),
            pltpu.VMEM((TPW, D), jnp.float32),
            pltpu.VMEM((TPW, D), jnp.float32),
            pltpu.SemaphoreType.DMA,
            pltpu.SemaphoreType.DMA,
            pltpu.SemaphoreType.DMA,
            pltpu.SemaphoreType.DMA,
        ],
        compiler_params=_sc_compiler_params(),
    )(ys, pos0, pos1, cw0, cw1)


def kernel(hidden_states, W_gate, w1, w3, w2):
    orig_shape = hidden_states.shape
    x = hidden_states.reshape(-1, D)
    wg_pad = jnp.pad(W_gate, ((0, 0), (0, EPAD - E)))

    route = _router(x, wg_pad)
    i0 = route[:, 0].astype(jnp.int32)
    i1 = route[:, 1].astype(jnp.int32)
    cw0 = route[:, 2]
    cw1 = route[:, 3]

    xs, pos0, pos1, blk = _dispatch(x, i0, i1)
    ys = _ffn(blk, xs, w1, w3, w2)
    out = _combine(ys, pos0, pos1, cw0, cw1)
    return out.reshape(orig_shape)


# speedup vs baseline: 1.6313x; 1.0190x over previous
import dataclasses

import jax
import jax.numpy as jnp
from jax import lax
from jax.experimental import pallas as pl
from jax.experimental.pallas import tpu as pltpu
from jax.experimental.pallas import tpu_sc as plsc

E = 8
TOPK = 2
D = 768
DFF = 2688
T = 2048

R = 512
CAP = T * TOPK + E * R
G_MAX = CAP // R
F = 3
DFB = DFF // F
EPAD = 128

NW = 32
TPW = T // NW
NB_SLOT = G_MAX


def _sc_compiler_params():
    cp = pltpu.CompilerParams()
    if "needs_layout_passes" in pltpu.CompilerParams.__dataclass_fields__:
        cp = dataclasses.replace(cp, needs_layout_passes=False)
    return cp


def _lane16():
    return lax.iota(jnp.int32, 16)


def _vlane(v, e):
    return jnp.sum(jnp.where(_lane16() == e, v, jnp.zeros_like(v)))


def _router_body(x_ref, wg_ref, o_ref):
    lg = jnp.dot(x_ref[...], wg_ref[...], preferred_element_type=jnp.float32)
    lane = lax.broadcasted_iota(jnp.int32, lg.shape, 1)
    valid = lane < E
    lgm = jnp.where(valid, lg, jnp.float32(-1e30))
    m = jnp.max(lgm, axis=1, keepdims=True)
    ex = jnp.where(valid, jnp.exp(lgm - m), 0.0)
    p = ex / jnp.sum(ex, axis=1, keepdims=True)
    m1 = jnp.max(p, axis=1, keepdims=True)
    i1 = jnp.min(jnp.where(p >= m1, lane, EPAD), axis=1, keepdims=True)
    p2 = jnp.where(lane == i1, jnp.float32(-1.0), p)
    m2 = jnp.max(p2, axis=1, keepdims=True)
    i2 = jnp.min(jnp.where(p2 >= m2, lane, EPAD), axis=1, keepdims=True)
    s = m1 + m2
    o_ref[...] = jnp.where(
        lane == 0, i1.astype(jnp.float32),
        jnp.where(lane == 1, i2.astype(jnp.float32),
                  jnp.where(lane == 2, m1 / s,
                            jnp.where(lane == 3, m2 / s, 0.0))))


def _router(x, wg_pad):
    return pl.pallas_call(
        _router_body,
        grid=(T // R,),
        in_specs=[
            pl.BlockSpec((R, D), lambda i: (i, 0)),
            pl.BlockSpec((D, EPAD), lambda i: (0, 0)),
        ],
        out_specs=pl.BlockSpec((R, EPAD), lambda i: (i, 0)),
        out_shape=jax.ShapeDtypeStruct((T, EPAD), jnp.float32),
    )(x, wg_pad)


def _dispatch_body(x_hbm, i0_hbm, i1_hbm,
                   xs_hbm, pos0_hbm, pos1_hbm, blk_hbm,
                   ivm0, ivm1, xrows, p0v, p1v, blkv):
    cid = lax.axis_index("c")
    sid = lax.axis_index("s")
    wid = sid * 2 + cid
    mybase = wid * TPW
    zeros16 = jnp.zeros((16,), jnp.int32)
    lanes = _lane16()

    pltpu.sync_copy(i0_hbm, ivm0)
    pltpu.sync_copy(i1_hbm, ivm1)
    pltpu.sync_copy(x_hbm.at[pl.ds(mybase, TPW)], xrows)

    def chunk_step(w, carry):
        totals, prefix = carry
        wv = jnp.full((16,), w, jnp.int32)
        is_prev = wv < jnp.full((16,), wid, jnp.int32)
        base = w * TPW
        for r in range(TPW // 16):
            for src in (ivm0, ivm1):
                v = src[pl.ds(base + r * 16, 16)]
                for e in range(E):
                    cnt = plsc.all_reduce_population_count(v == e)
                    delta = jnp.where(lanes == e, cnt, zeros16)
                    totals = totals + delta
                    prefix = prefix + jnp.where(is_prev, delta, zeros16)
        return totals, prefix

    totals, prefix = lax.fori_loop(0, NW, chunk_step, (zeros16, zeros16))

    padded = ((totals + (R - 1)) // R) * R
    incl = plsc.cumsum(padded)
    seg_start = incl - padded
    nb = _vlane(incl, E - 1) // R

    running = seg_start + prefix
    for r in range(TPW // 16):
        for src, dst in ((ivm0, p0v), (ivm1, p1v)):
            v = src[pl.ds(mybase + r * 16, 16)]
            posv = zeros16
            for e in range(E):
                m = v == e
                ones = jnp.where(m, jnp.ones((16,), jnp.int32), zeros16)
                rank = plsc.cumsum(ones) - 1
                b_e = _vlane(running, e)
                posv = jnp.where(m, b_e + rank, posv)
                running = running + jnp.where(lanes == e, jnp.sum(ones), 0)
            dst[0, pl.ds(r * 16, 16)] = posv

    pltpu.sync_copy(p0v.at[0], pos0_hbm.at[pl.ds(mybase, TPW)])
    pltpu.sync_copy(p1v.at[0], pos1_hbm.at[pl.ds(mybase, TPW)])
    pltpu.sync_copy(xrows, xs_hbm.at[p0v.at[0]])
    pltpu.sync_copy(xrows, xs_hbm.at[p1v.at[0]])

    @pl.when(wid == 0)
    def _():
        for half in range(2):
            bidx = lanes + half * 16
            beff = jnp.minimum(bidx, nb - 1)
            posb = beff * R
            ev = zeros16
            for e in range(E):
                s_e = _vlane(seg_start, e)
                p_e = _vlane(padded, e)
                m = (posb >= s_e) & (posb < s_e + p_e)
                ev = jnp.where(m, jnp.full((16,), e, jnp.int32), ev)
            if half == 1:
                ev = jnp.where(lanes == NB_SLOT - 16,
                               jnp.full((16,), nb, jnp.int32), ev)
            blkv[0, pl.ds(half * 16, 16)] = ev
        pltpu.sync_copy(blkv.at[0], blk_hbm)


def _dispatch(x, i0, i1):
    mesh = plsc.VectorSubcoreMesh(core_axis_name="c", subcore_axis_name="s")
    return pl.kernel(
        _dispatch_body,
        out_type=(
            jax.ShapeDtypeStruct((CAP, D), jnp.float32),
            jax.ShapeDtypeStruct((T,), jnp.int32),
            jax.ShapeDtypeStruct((T,), jnp.int32),
            jax.ShapeDtypeStruct((32,), jnp.int32),
        ),
        mesh=mesh,
        scratch_types=[
            pltpu.VMEM((T,), jnp.int32),
            pltpu.VMEM((T,), jnp.int32),
            pltpu.VMEM((TPW, D), jnp.float32),
            pltpu.VMEM((1, TPW), jnp.int32),
            pltpu.VMEM((1, TPW), jnp.int32),
            pltpu.VMEM((1, 32), jnp.int32),
        ],
        compiler_params=_sc_compiler_params(),
    )(x, i0, i1)


def _ffn_body(s_ref, xs_ref, w1_ref, w3_ref, w2_ref, ys_ref, acc_ref,
              w1c_ref, w3c_ref, w2c_ref, xsb_ref):
    f = pl.program_id(0)
    b = pl.program_id(1)

    @pl.when(b < s_ref[NB_SLOT])
    def _():
        bm1 = jnp.maximum(b - 1, 0)

        @pl.when((b == 0) | (s_ref[b] != s_ref[bm1]))
        def _():
            w1c_ref[...] = w1_ref[0].astype(jnp.bfloat16)
            w3c_ref[...] = w3_ref[0].astype(jnp.bfloat16)
            w2c_ref[...] = w2_ref[0].astype(jnp.bfloat16)

        sl = pl.ds(b * R, R)

        @pl.when(f == 0)
        def _():
            xsb_ref[sl, :] = xs_ref[...].astype(jnp.bfloat16)

        xb = xsb_ref[sl, :]
        w1e = w1c_ref[...]
        w3e = w3c_ref[...]
        w2e = w2c_ref[...]
        nt = (((1,), (1,)), ((), ()))
        h1 = lax.dot_general(xb, w1e, nt,
                             preferred_element_type=jnp.float32).astype(jnp.bfloat16)
        h3 = lax.dot_general(xb, w3e, nt,
                             preferred_element_type=jnp.float32).astype(jnp.bfloat16)
        g = h1 * jax.nn.sigmoid(h1) * h3
        y = lax.dot_general(g, w2e, nt, preferred_element_type=jnp.float32)

        @pl.when(f == 0)
        def _():
            acc_ref[sl, :] = y.astype(jnp.bfloat16)

        @pl.when(f > 0)
        def _():
            acc_ref[sl, :] = acc_ref[sl, :] + y.astype(jnp.bfloat16)

        @pl.when(f == F - 1)
        def _():
            ys_ref[...] = (acc_ref[sl, :]).astype(jnp.float32)


def _ffn(blk, xs, w1, w3, w2):
    grid_spec = pltpu.PrefetchScalarGridSpec(
        num_scalar_prefetch=1,
        grid=(F, G_MAX),
        in_specs=[
            pl.BlockSpec((R, D),
                         lambda f, b, s: (jnp.where(
                             f == 0, jnp.minimum(b, s[NB_SLOT] - 1),
                             s[NB_SLOT] - 1), 0)),
            pl.BlockSpec((1, DFB, D), lambda f, b, s: (s[b], f, 0)),
            pl.BlockSpec((1, DFB, D), lambda f, b, s: (s[b], f, 0)),
            pl.BlockSpec((1, D, DFB), lambda f, b, s: (s[b], 0, f)),
        ],
        out_specs=pl.BlockSpec((R, D),
                               lambda f, b, s: (jnp.where(f == F - 1, b, 0), 0)),
        scratch_shapes=[
            pltpu.VMEM((CAP, D), jnp.bfloat16),
            pltpu.VMEM((DFB, D), jnp.bfloat16),
            pltpu.VMEM((DFB, D), jnp.bfloat16),
            pltpu.VMEM((D, DFB), jnp.bfloat16),
            pltpu.VMEM((CAP, D), jnp.bfloat16),
        ],
    )
    return pl.pallas_call(
        _ffn_body,
        grid_spec=grid_spec,
        out_shape=jax.ShapeDtypeStruct((CAP, D), jnp.float32),
    )(blk, xs, w1, w3, w2)


def _combine_body(ys_hbm, pos0_hbm, pos1_hbm, cw0_hbm, cw1_hbm, out_hbm,
                  p0v, p1v, w0v, w1v, buf0, buf1, sem0, sem1, sem2, sem3):
    cid = lax.axis_index("c")
    sid = lax.axis_index("s")
    wid = sid * 2 + cid
    base = wid * TPW

    cp0 = pltpu.async_copy(pos0_hbm.at[pl.ds(base, TPW)], p0v.at[0], sem0)
    cp1 = pltpu.async_copy(pos1_hbm.at[pl.ds(base, TPW)], p1v.at[0], sem1)
    cwa = pltpu.async_copy(cw0_hbm.at[pl.ds(base, TPW)], w0v, sem2)
    cwc = pltpu.async_copy(cw1_hbm.at[pl.ds(base, TPW)], w1v, sem3)
    cp0.wait()
    g0 = pltpu.async_copy(ys_hbm.at[p0v.at[0]], buf0, sem0)
    cp1.wait()
    g1 = pltpu.async_copy(ys_hbm.at[p1v.at[0]], buf1, sem1)
    cwa.wait()
    cwc.wait()
    g0.wait()
    g1.wait()

    for tq in range(TPW // 16):
        wa = w0v[pl.ds(tq * 16, 16)]
        wc = w1v[pl.ds(tq * 16, 16)]
        for k in range(16):
            t = tq * 16 + k

            @plsc.parallel_loop(0, D // 16, unroll=4)
            def _(dd, t=t, a=wa[k], c=wc[k]):
                sl = pl.ds(dd * 16, 16)
                buf0[t, sl] = a * buf0[t, sl] + c * buf1[t, sl]

    pltpu.sync_copy(buf0, out_hbm.at[pl.ds(base, TPW)])


def _combine(ys, pos0, pos1, cw0, cw1):
    mesh = plsc.VectorSubcoreMesh(core_axis_name="c", subcore_axis_name="s")
    return pl.kernel(
        _combine_body,
        out_type=jax.ShapeDtypeStruct((T, D), jnp.float32),
        mesh=mesh,
        scratch_types=[
            pltpu.VMEM((1, TPW), jnp.int32),
            pltpu.VMEM((1, TPW), jnp.int32),
            pltpu.VMEM((TPW,), jnp.float32),
            pltpu.VMEM((TPW,), jnp.float32),
            pltpu.VMEM((TPW, D), jnp.float32),
            pltpu.VMEM((TPW, D), jnp.float32),
            pltpu.SemaphoreType.DMA,
            pltpu.SemaphoreType.DMA,
            pltpu.SemaphoreType.DMA,
            pltpu.SemaphoreType.DMA,
        ],
        compiler_params=_sc_compiler_params(),
    )(ys, pos0, pos1, cw0, cw1)


def kernel(hidden_states, W_gate, w1, w3, w2):
    orig_shape = hidden_states.shape
    x = hidden_states.reshape(-1, D)
    wg_pad = jnp.pad(W_gate, ((0, 0), (0, EPAD - E)))

    route = _router(x, wg_pad)
    i0 = route[:, 0].astype(jnp.int32)
    i1 = route[:, 1].astype(jnp.int32)
    cw0 = route[:, 2]
    cw1 = route[:, 3]

    xs, pos0, pos1, blk = _dispatch(x, i0, i1)
    ys = _ffn(blk, xs, w1, w3, w2)
    out = _combine(ys, pos0, pos1, cw0, cw1)
    return out.reshape(orig_shape)
